# baseline (speedup 1.0000x reference)
import functools

import jax
import jax.numpy as jnp
from jax import lax
from jax.experimental import pallas as pl
from jax.experimental.pallas import tpu as pltpu
from jax.experimental.pallas import tpu_sc as plsc

_BATCH = 1024
_TS = 1000
_NW = 16
_BPW = _BATCH // _NW
_L = 16


@functools.partial(
    pl.kernel,
    mesh=plsc.VectorSubcoreMesh(core_axis_name="c", subcore_axis_name="s",
                                num_cores=1),
    compiler_params=pltpu.CompilerParams(
        needs_layout_passes=False,
        skip_device_barrier=True,
        disable_bounds_checks=True,
        disable_semaphore_checks=True,
    ),
    out_type=(
        jax.ShapeDtypeStruct((_BATCH,), jnp.float32),
        jax.ShapeDtypeStruct((_BATCH,), jnp.float32),
    ),
    scratch_types=[
        pltpu.VMEM((_BPW,), jnp.int32),
        pltpu.VMEM((1024,), jnp.float32),
        pltpu.VMEM((1024,), jnp.float32),
        pltpu.VMEM((_BPW,), jnp.float32),
        pltpu.VMEM((_BPW,), jnp.float32),
        pltpu.SemaphoreType.DMA,
        pltpu.SemaphoreType.DMA,
    ],
)
def _gather_bt_at(t_hbm, beta_hbm, alpha_hbm, beta_out, alpha_out,
                  idx_v, bt_v, at_v, bo_v, ao_v, sem_in, sem_out):
    base = lax.axis_index("s") * _BPW
    ci = pltpu.async_copy(t_hbm.at[pl.ds(base, _BPW)], idx_v, sem_in)
    cb = pltpu.async_copy(beta_hbm, bt_v.at[pl.ds(0, _TS)], sem_in)
    ca = pltpu.async_copy(alpha_hbm, at_v.at[pl.ds(0, _TS)], sem_in)
    ci.wait()
    cb.wait()
    ca.wait()

    def body_b(j, _):
        iv = idx_v[pl.ds(j * _L, _L)]
        bo_v[pl.ds(j * _L, _L)] = plsc.load_gather(bt_v, [iv])
        return 0

    def body_a(j, _):
        iv = idx_v[pl.ds(j * _L, _L)]
        ao_v[pl.ds(j * _L, _L)] = plsc.load_gather(at_v, [iv])
        return 0

    lax.fori_loop(0, _BPW // _L, body_b, 0, unroll=False)
    ob = pltpu.async_copy(bo_v, beta_out.at[pl.ds(base, _BPW)], sem_out)
    lax.fori_loop(0, _BPW // _L, body_a, 0, unroll=False)
    oa = pltpu.async_copy(ao_v, alpha_out.at[pl.ds(base, _BPW)], sem_out)
    ob.wait()
    oa.wait()


def kernel(x, t, beta, alpha):
    return _gather_bt_at(t, beta, alpha)


# speedup vs baseline: 1.0046x; 1.0046x over previous
import functools

import jax
import jax.numpy as jnp
from jax import lax
from jax.experimental import pallas as pl
from jax.experimental.pallas import tpu as pltpu
from jax.experimental.pallas import tpu_sc as plsc

_BATCH = 1024
_TS = 1000
_NW = 16
_BPW = _BATCH // _NW
_L = 16


@functools.partial(
    pl.kernel,
    mesh=plsc.VectorSubcoreMesh(core_axis_name="c", subcore_axis_name="s",
                                num_cores=1),
    compiler_params=pltpu.CompilerParams(
        needs_layout_passes=False,
        skip_device_barrier=True,
        disable_bounds_checks=True,
        disable_semaphore_checks=True,
    ),
    out_type=(
        jax.ShapeDtypeStruct((_BATCH,), jnp.float32),
        jax.ShapeDtypeStruct((_BATCH,), jnp.float32),
    ),
    scratch_types=[
        pltpu.VMEM((_BPW,), jnp.int32),
        pltpu.VMEM((1024,), jnp.float32),
        pltpu.VMEM((1024,), jnp.float32),
        pltpu.VMEM((_BPW,), jnp.float32),
        pltpu.VMEM((_BPW,), jnp.float32),
        pltpu.SemaphoreType.DMA,
        pltpu.SemaphoreType.DMA,
    ],
)
def _gather_bt_at(t_hbm, beta_hbm, alpha_hbm, beta_out, alpha_out,
                  idx_v, bt_v, at_v, bo_v, ao_v, sem_in, sem_out):
    base = lax.axis_index("s") * _BPW
    ci = pltpu.async_copy(t_hbm.at[pl.ds(base, _BPW)], idx_v, sem_in)
    cb = pltpu.async_copy(beta_hbm, bt_v.at[pl.ds(0, _TS)], sem_in)
    ca = pltpu.async_copy(alpha_hbm, at_v.at[pl.ds(0, _TS)], sem_in)
    ci.wait()
    cb.wait()
    ca.wait()

    def body(j, _):
        iv = idx_v[pl.ds(j * _L, _L)]
        bo_v[pl.ds(j * _L, _L)] = plsc.load_gather(bt_v, [iv])
        ao_v[pl.ds(j * _L, _L)] = plsc.load_gather(at_v, [iv])
        return 0

    lax.fori_loop(0, _BPW // _L, body, 0, unroll=False)
    ob = pltpu.async_copy(bo_v, beta_out.at[pl.ds(base, _BPW)], sem_out)
    oa = pltpu.async_copy(ao_v, alpha_out.at[pl.ds(base, _BPW)], sem_out)
    ob.wait()
    oa.wait()


def kernel(x, t, beta, alpha):
    return _gather_bt_at(t, beta, alpha)
